# baseline (speedup 1.0000x reference)
import jax
import jax.numpy as jnp
from jax import lax
from jax.experimental import pallas as pl
from jax.experimental.pallas import tpu as pltpu
from jax.experimental.pallas import tpu_sc as plsc

N_NODES = 50000
D = 128
FILTER = 10

NW = 32
GROUP = 112
IDX_PER_ROW = 11
B_PAD = 50176
ROWS_PER_W = B_PAD // NW
GROUPS_PER_W = ROWS_PER_W // GROUP
IDX_PER_GROUP = GROUP * IDX_PER_ROW

BM = 448


def _sc_body(feat_hbm, idx_hbm, out_hbm, idx_v, self_v, acc_v, sem):
    wid = lax.axis_index("s") * 2 + (1 - lax.axis_index("c"))
    zeros16 = jnp.zeros((16,), jnp.float32)

    def group_body(g, carry):
        idx_base = (wid * GROUPS_PER_W + g) * IDX_PER_GROUP
        pltpu.sync_copy(idx_hbm.at[pl.ds(idx_base, IDX_PER_GROUP)], idx_v)

        def zero_body(r, c):
            for v in range(D // 16):
                acc_v[r, pl.ds(v * 16, 16)] = zeros16
            return c

        lax.fori_loop(0, GROUP, zero_body, 0)

        copies = [
            pltpu.async_copy(
                feat_hbm.at[idx_v.at[pl.ds(0, GROUP)]], self_v, sem
            )
        ] + [
            pltpu.async_copy(
                feat_hbm.at[idx_v.at[pl.ds(j * GROUP, GROUP)]],
                acc_v,
                sem,
                add=True,
            )
            for j in range(1, IDX_PER_ROW)
        ]
        for c in copies:
            c.wait()

        row0 = wid * ROWS_PER_W + g * GROUP
        pltpu.sync_copy(self_v, out_hbm.at[pl.ds(row0, GROUP)])
        pltpu.sync_copy(acc_v, out_hbm.at[pl.ds(B_PAD + row0, GROUP)])
        return carry

    lax.fori_loop(0, GROUPS_PER_W, group_body, 0)


def _sc_gather(features, idx_hbm):
    mesh = plsc.VectorSubcoreMesh(core_axis_name="c", subcore_axis_name="s")
    return pl.kernel(
        _sc_body,
        out_type=jax.ShapeDtypeStruct((2 * B_PAD, D), jnp.float32),
        mesh=mesh,
        scratch_types=[
            pltpu.VMEM((IDX_PER_GROUP,), jnp.int32),
            pltpu.VMEM((GROUP, D), jnp.float32),
            pltpu.VMEM((GROUP, D), jnp.float32),
            pltpu.SemaphoreType.DMA,
        ],
    )(features, idx_hbm)


def _matmul_body(s_ref, n_ref, w_ref, o_ref):
    ps = lax.dot_general(
        s_ref[...], w_ref[0],
        dimension_numbers=(((1,), (0,)), ((), ())),
        preferred_element_type=jnp.float32,
    )
    pn = lax.dot_general(
        n_ref[...], w_ref[1],
        dimension_numbers=(((1,), (0,)), ((), ())),
        preferred_element_type=jnp.float32,
    )
    o_ref[...] = jnp.maximum(ps + pn, 0.0)


def _fused_matmul(gathered, w_stack):
    nblk = B_PAD // BM
    return pl.pallas_call(
        _matmul_body,
        grid=(nblk,),
        in_specs=[
            pl.BlockSpec((BM, D), lambda i: (i, 0)),
            pl.BlockSpec((BM, D), lambda i: (nblk + i, 0)),
            pl.BlockSpec((2, D, D), lambda i: (0, 0, 0)),
        ],
        out_specs=pl.BlockSpec((BM, D), lambda i: (i, 0)),
        out_shape=jax.ShapeDtypeStruct((B_PAD, D), jnp.float32),
    )(gathered, gathered, w_stack)


def kernel(nodes, neigh_idx, features, detaching_weight):
    w_top = detaching_weight[:D]
    w_bot = detaching_weight[D:] * (1.0 / FILTER)
    w_stack = jnp.stack([w_top, w_bot])

    nodes32 = nodes.astype(jnp.int32)
    idx11 = jnp.concatenate(
        [nodes32[:, None], neigh_idx.astype(jnp.int32)], axis=1
    )
    idx_pad = jnp.pad(idx11, ((0, B_PAD - N_NODES), (0, 0)))
    idx_t = jnp.swapaxes(
        idx_pad.reshape(NW * GROUPS_PER_W, GROUP, IDX_PER_ROW), 1, 2
    )
    idx_hbm = idx_t.reshape(-1)

    gathered = _sc_gather(features, idx_hbm)
    out_pad = _fused_matmul(gathered, w_stack)
    return out_pad[:N_NODES]


# speedup vs baseline: 1.4159x; 1.4159x over previous
import jax
import jax.numpy as jnp
from jax import lax
from jax.experimental import pallas as pl
from jax.experimental.pallas import tpu as pltpu
from jax.experimental.pallas import tpu_sc as plsc

N_NODES = 50000
D = 128
FILTER = 10

NW = 32
GROUP = 112
IDX_PER_ROW = 11
B_PAD = 50176
ROWS_PER_W = B_PAD // NW
GROUPS_PER_W = ROWS_PER_W // GROUP
IDX_PER_GROUP = GROUP * IDX_PER_ROW

BM = 448


def _sc_body(feat_hbm, idx_hbm, out_hbm, idx_v, self_v, acc_v, sem):
    wid = lax.axis_index("s") * 2 + lax.axis_index("c")
    zeros16 = jnp.zeros((16,), jnp.float32)

    def group_body(g, carry):
        idx_base = (wid * GROUPS_PER_W + g) * IDX_PER_GROUP
        pltpu.sync_copy(idx_hbm.at[pl.ds(idx_base, IDX_PER_GROUP)], idx_v)

        def zero_body(r, c):
            for v in range(D // 16):
                acc_v[r, pl.ds(v * 16, 16)] = zeros16
            return c

        lax.fori_loop(0, GROUP, zero_body, 0)

        copies = [
            pltpu.async_copy(
                feat_hbm.at[idx_v.at[pl.ds(0, GROUP)]], self_v, sem
            )
        ] + [
            pltpu.async_copy(
                feat_hbm.at[idx_v.at[pl.ds(j * GROUP, GROUP)]],
                acc_v,
                sem,
                add=True,
            )
            for j in range(1, IDX_PER_ROW)
        ]
        for c in copies:
            c.wait()

        row0 = wid * ROWS_PER_W + g * GROUP
        pltpu.sync_copy(self_v, out_hbm.at[pl.ds(row0, GROUP)])
        pltpu.sync_copy(acc_v, out_hbm.at[pl.ds(B_PAD + row0, GROUP)])
        return carry

    lax.fori_loop(0, GROUPS_PER_W, group_body, 0)


def _sc_gather(features, idx_hbm):
    mesh = plsc.VectorSubcoreMesh(core_axis_name="c", subcore_axis_name="s")
    return pl.kernel(
        _sc_body,
        out_type=jax.ShapeDtypeStruct((2 * B_PAD, D), jnp.float32),
        mesh=mesh,
        scratch_types=[
            pltpu.VMEM((IDX_PER_GROUP,), jnp.int32),
            pltpu.VMEM((GROUP, D), jnp.float32),
            pltpu.VMEM((GROUP, D), jnp.float32),
            pltpu.SemaphoreType.DMA,
        ],
    )(features, idx_hbm)


def _matmul_body(s_ref, n_ref, w_ref, o_ref):
    ps = lax.dot_general(
        s_ref[...], w_ref[0],
        dimension_numbers=(((1,), (0,)), ((), ())),
        preferred_element_type=jnp.float32,
    )
    pn = lax.dot_general(
        n_ref[...], w_ref[1],
        dimension_numbers=(((1,), (0,)), ((), ())),
        preferred_element_type=jnp.float32,
    )
    o_ref[...] = jnp.maximum(ps + pn, 0.0)


def _fused_matmul(gathered, w_stack):
    nblk = B_PAD // BM
    return pl.pallas_call(
        _matmul_body,
        grid=(nblk,),
        in_specs=[
            pl.BlockSpec((BM, D), lambda i: (i, 0)),
            pl.BlockSpec((BM, D), lambda i: (nblk + i, 0)),
            pl.BlockSpec((2, D, D), lambda i: (0, 0, 0)),
        ],
        out_specs=pl.BlockSpec((BM, D), lambda i: (i, 0)),
        out_shape=jax.ShapeDtypeStruct((B_PAD, D), jnp.float32),
    )(gathered, gathered, w_stack)


def kernel(nodes, neigh_idx, features, detaching_weight):
    w_top = detaching_weight[:D]
    w_bot = detaching_weight[D:] * (1.0 / FILTER)
    w_stack = jnp.stack([w_top, w_bot])

    nodes32 = nodes.astype(jnp.int32)
    idx11 = jnp.concatenate(
        [nodes32[:, None], neigh_idx.astype(jnp.int32)], axis=1
    )
    pad_n = B_PAD - N_NODES
    pad_idx = (jnp.arange(pad_n * IDX_PER_ROW, dtype=jnp.int32) * 283 % N_NODES)
    idx_pad = jnp.concatenate(
        [idx11, pad_idx.reshape(pad_n, IDX_PER_ROW)], axis=0
    )
    idx_t = jnp.swapaxes(
        idx_pad.reshape(NW * GROUPS_PER_W, GROUP, IDX_PER_ROW), 1, 2
    )
    idx_hbm = idx_t.reshape(-1)

    gathered = _sc_gather(features, idx_hbm)
    out_pad = _fused_matmul(gathered, w_stack)
    return out_pad[:N_NODES]


# speedup vs baseline: 1.7223x; 1.2163x over previous
import jax
import jax.numpy as jnp
from jax import lax
from jax.experimental import pallas as pl
from jax.experimental.pallas import tpu as pltpu
from jax.experimental.pallas import tpu_sc as plsc

N_NODES = 50000
D = 128
FILTER = 10

NW = 32
GROUP = 112
IDX_PER_ROW = 11
B_PAD = 50176
ROWS_PER_W = B_PAD // NW
GROUPS_PER_W = ROWS_PER_W // GROUP
IDX_PER_GROUP = GROUP * IDX_PER_ROW

BM = 1000


def _sc_body(feat_hbm, idx_hbm, out_hbm, idx_v, self_v, acc_v, sem):
    wid = lax.axis_index("s") * 2 + lax.axis_index("c")
    zeros16 = jnp.zeros((16,), jnp.float32)

    def group_body(g, carry):
        idx_base = (wid * GROUPS_PER_W + g) * IDX_PER_GROUP
        pltpu.sync_copy(idx_hbm.at[pl.ds(idx_base, IDX_PER_GROUP)], idx_v)

        def zero_body(r, c):
            for v in range(D // 16):
                acc_v[r, pl.ds(v * 16, 16)] = zeros16
            return c

        lax.fori_loop(0, GROUP, zero_body, 0)

        copies = [
            pltpu.async_copy(
                feat_hbm.at[idx_v.at[pl.ds(0, GROUP)]], self_v, sem
            )
        ] + [
            pltpu.async_copy(
                feat_hbm.at[idx_v.at[pl.ds(j * GROUP, GROUP)]],
                acc_v,
                sem,
                add=True,
            )
            for j in range(1, IDX_PER_ROW)
        ]
        for c in copies:
            c.wait()

        row0 = wid * ROWS_PER_W + g * GROUP
        pltpu.sync_copy(self_v, out_hbm.at[pl.ds(row0, GROUP)])
        pltpu.sync_copy(acc_v, out_hbm.at[pl.ds(B_PAD + row0, GROUP)])
        return carry

    lax.fori_loop(0, GROUPS_PER_W, group_body, 0)


def _sc_gather(features, idx_hbm):
    mesh = plsc.VectorSubcoreMesh(core_axis_name="c", subcore_axis_name="s")
    return pl.kernel(
        _sc_body,
        out_type=jax.ShapeDtypeStruct((2 * B_PAD, D), jnp.float32),
        mesh=mesh,
        scratch_types=[
            pltpu.VMEM((IDX_PER_GROUP,), jnp.int32),
            pltpu.VMEM((GROUP, D), jnp.float32),
            pltpu.VMEM((GROUP, D), jnp.float32),
            pltpu.SemaphoreType.DMA,
        ],
    )(features, idx_hbm)


def _matmul_body(s_ref, n_ref, w_ref, o_ref):
    ps = lax.dot_general(
        s_ref[0], w_ref[0],
        dimension_numbers=(((1,), (0,)), ((), ())),
        preferred_element_type=jnp.float32,
    )
    pn = lax.dot_general(
        n_ref[0], w_ref[1],
        dimension_numbers=(((1,), (0,)), ((), ())),
        preferred_element_type=jnp.float32,
    )
    o_ref[...] = jnp.maximum(ps + pn, 0.0)


def _fused_matmul(gathered, w_stack):
    nblk = N_NODES // BM
    g3 = gathered.reshape(2, B_PAD, D)
    return pl.pallas_call(
        _matmul_body,
        grid=(nblk,),
        in_specs=[
            pl.BlockSpec((1, BM, D), lambda i: (0, i, 0)),
            pl.BlockSpec((1, BM, D), lambda i: (1, i, 0)),
            pl.BlockSpec((2, D, D), lambda i: (0, 0, 0)),
        ],
        out_specs=pl.BlockSpec((BM, D), lambda i: (i, 0)),
        out_shape=jax.ShapeDtypeStruct((N_NODES, D), jnp.float32),
    )(g3, g3, w_stack)


def kernel(nodes, neigh_idx, features, detaching_weight):
    w_top = detaching_weight[:D]
    w_bot = detaching_weight[D:] * (1.0 / FILTER)
    w_stack = jnp.stack([w_top, w_bot])

    nodes32 = nodes.astype(jnp.int32)
    idx11 = jnp.concatenate(
        [nodes32[:, None], neigh_idx.astype(jnp.int32)], axis=1
    )
    pad_n = B_PAD - N_NODES
    pad_idx = (jnp.arange(pad_n * IDX_PER_ROW, dtype=jnp.int32) * 283 % N_NODES)
    idx_pad = jnp.concatenate(
        [idx11, pad_idx.reshape(pad_n, IDX_PER_ROW)], axis=0
    )
    idx_t = jnp.swapaxes(
        idx_pad.reshape(NW * GROUPS_PER_W, GROUP, IDX_PER_ROW), 1, 2
    )
    idx_hbm = idx_t.reshape(-1)

    gathered = _sc_gather(features, idx_hbm)
    return _fused_matmul(gathered, w_stack)


# speedup vs baseline: 1.8664x; 1.0837x over previous
import jax
import jax.numpy as jnp
from jax import lax
from jax.experimental import pallas as pl
from jax.experimental.pallas import tpu as pltpu
from jax.experimental.pallas import tpu_sc as plsc

N_NODES = 50000
D = 128
FILTER = 10

NW = 32
GROUP = 112
IDX_PER_ROW = 11
B_PAD = 50176
ROWS_PER_W = B_PAD // NW
GROUPS_PER_W = ROWS_PER_W // GROUP
IDX_PER_GROUP = GROUP * IDX_PER_ROW

BM = 2000


def _sc_body(feat_hbm, idx_hbm, out_hbm, idx_v, self_v, acc_v, sem):
    wid = lax.axis_index("s") * 2 + lax.axis_index("c")
    zeros16 = jnp.zeros((16,), jnp.float32)

    def group_body(g, carry):
        idx_base = (wid * GROUPS_PER_W + g) * IDX_PER_GROUP
        pltpu.sync_copy(idx_hbm.at[pl.ds(idx_base, IDX_PER_GROUP)], idx_v)

        def zero_body(r, c):
            for v in range(D // 16):
                acc_v[r, pl.ds(v * 16, 16)] = zeros16
            return c

        lax.fori_loop(0, GROUP, zero_body, 0)

        copies = [
            pltpu.async_copy(
                feat_hbm.at[idx_v.at[pl.ds(0, GROUP)]], self_v, sem
            )
        ] + [
            pltpu.async_copy(
                feat_hbm.at[idx_v.at[pl.ds(j * GROUP, GROUP)]],
                acc_v,
                sem,
                add=True,
            )
            for j in range(1, IDX_PER_ROW)
        ]
        for c in copies:
            c.wait()

        row0 = wid * ROWS_PER_W + g * GROUP
        pltpu.sync_copy(self_v, out_hbm.at[pl.ds(row0, GROUP)])
        pltpu.sync_copy(acc_v, out_hbm.at[pl.ds(B_PAD + row0, GROUP)])
        return carry

    lax.fori_loop(0, GROUPS_PER_W, group_body, 0)


def _sc_gather(features, idx_hbm):
    mesh = plsc.VectorSubcoreMesh(core_axis_name="c", subcore_axis_name="s")
    return pl.kernel(
        _sc_body,
        out_type=jax.ShapeDtypeStruct((2 * B_PAD, D), jnp.float32),
        mesh=mesh,
        scratch_types=[
            pltpu.VMEM((IDX_PER_GROUP,), jnp.int32),
            pltpu.VMEM((GROUP, D), jnp.float32),
            pltpu.VMEM((GROUP, D), jnp.float32),
            pltpu.SemaphoreType.DMA,
        ],
    )(features, idx_hbm)


def _matmul_body(s_ref, n_ref, w_ref, o_ref):
    ps = lax.dot_general(
        s_ref[0], w_ref[0],
        dimension_numbers=(((1,), (0,)), ((), ())),
        preferred_element_type=jnp.float32,
    )
    pn = lax.dot_general(
        n_ref[0], w_ref[1],
        dimension_numbers=(((1,), (0,)), ((), ())),
        preferred_element_type=jnp.float32,
    )
    o_ref[...] = jnp.maximum(ps + pn, 0.0)


def _fused_matmul(gathered, w_stack):
    nblk = N_NODES // BM
    g3 = gathered.reshape(2, B_PAD, D)
    return pl.pallas_call(
        _matmul_body,
        grid=(nblk,),
        in_specs=[
            pl.BlockSpec((1, BM, D), lambda i: (0, i, 0)),
            pl.BlockSpec((1, BM, D), lambda i: (1, i, 0)),
            pl.BlockSpec((2, D, D), lambda i: (0, 0, 0)),
        ],
        out_specs=pl.BlockSpec((BM, D), lambda i: (i, 0)),
        out_shape=jax.ShapeDtypeStruct((N_NODES, D), jnp.float32),
    )(g3, g3, w_stack)


def kernel(nodes, neigh_idx, features, detaching_weight):
    w_top = detaching_weight[:D]
    w_bot = detaching_weight[D:] * (1.0 / FILTER)
    w_stack = jnp.stack([w_top, w_bot])

    nodes32 = nodes.astype(jnp.int32)
    idx11 = jnp.concatenate(
        [nodes32[:, None], neigh_idx.astype(jnp.int32)], axis=1
    )
    pad_n = B_PAD - N_NODES
    pad_idx = (jnp.arange(pad_n * IDX_PER_ROW, dtype=jnp.int32) * 283 % N_NODES)
    idx_pad = jnp.concatenate(
        [idx11, pad_idx.reshape(pad_n, IDX_PER_ROW)], axis=0
    )
    idx_t = jnp.swapaxes(
        idx_pad.reshape(NW * GROUPS_PER_W, GROUP, IDX_PER_ROW), 1, 2
    )
    idx_hbm = idx_t.reshape(-1)

    gathered = _sc_gather(features, idx_hbm)
    return _fused_matmul(gathered, w_stack)


# speedup vs baseline: 1.9170x; 1.0271x over previous
import jax
import jax.numpy as jnp
from jax import lax
from jax.experimental import pallas as pl
from jax.experimental.pallas import tpu as pltpu
from jax.experimental.pallas import tpu_sc as plsc

N_NODES = 50000
D = 128
FILTER = 10

NW = 32
GROUP = 112
IDX_PER_ROW = 11
B_PAD = 50176
ROWS_PER_W = B_PAD // NW
GROUPS_PER_W = ROWS_PER_W // GROUP
IDX_PER_GROUP = GROUP * IDX_PER_ROW

BM = 5000


def _sc_body(feat_hbm, idx_hbm, out_hbm, idx_v, self_v, acc_v, sem):
    wid = lax.axis_index("s") * 2 + lax.axis_index("c")
    zeros16 = jnp.zeros((16,), jnp.float32)

    def group_body(g, carry):
        idx_base = (wid * GROUPS_PER_W + g) * IDX_PER_GROUP
        pltpu.sync_copy(idx_hbm.at[pl.ds(idx_base, IDX_PER_GROUP)], idx_v)

        def zero_body(r, c):
            for v in range(D // 16):
                acc_v[r, pl.ds(v * 16, 16)] = zeros16
            return c

        lax.fori_loop(0, GROUP, zero_body, 0)

        copies = [
            pltpu.async_copy(
                feat_hbm.at[idx_v.at[pl.ds(0, GROUP)]], self_v, sem
            )
        ] + [
            pltpu.async_copy(
                feat_hbm.at[idx_v.at[pl.ds(j * GROUP, GROUP)]],
                acc_v,
                sem,
                add=True,
            )
            for j in range(1, IDX_PER_ROW)
        ]
        for c in copies:
            c.wait()

        row0 = wid * ROWS_PER_W + g * GROUP
        pltpu.sync_copy(self_v, out_hbm.at[pl.ds(row0, GROUP)])
        pltpu.sync_copy(acc_v, out_hbm.at[pl.ds(B_PAD + row0, GROUP)])
        return carry

    lax.fori_loop(0, GROUPS_PER_W, group_body, 0)


def _sc_gather(features, idx_hbm):
    mesh = plsc.VectorSubcoreMesh(core_axis_name="c", subcore_axis_name="s")
    return pl.kernel(
        _sc_body,
        out_type=jax.ShapeDtypeStruct((2 * B_PAD, D), jnp.float32),
        mesh=mesh,
        scratch_types=[
            pltpu.VMEM((IDX_PER_GROUP,), jnp.int32),
            pltpu.VMEM((GROUP, D), jnp.float32),
            pltpu.VMEM((GROUP, D), jnp.float32),
            pltpu.SemaphoreType.DMA,
        ],
    )(features, idx_hbm)


def _matmul_body(s_ref, n_ref, w_ref, o_ref):
    ps = lax.dot_general(
        s_ref[0], w_ref[0],
        dimension_numbers=(((1,), (0,)), ((), ())),
        preferred_element_type=jnp.float32,
    )
    pn = lax.dot_general(
        n_ref[0], w_ref[1],
        dimension_numbers=(((1,), (0,)), ((), ())),
        preferred_element_type=jnp.float32,
    )
    o_ref[...] = jnp.maximum(ps + pn, 0.0)


def _fused_matmul(gathered, w_stack):
    nblk = N_NODES // BM
    g3 = gathered.reshape(2, B_PAD, D)
    return pl.pallas_call(
        _matmul_body,
        grid=(nblk,),
        in_specs=[
            pl.BlockSpec((1, BM, D), lambda i: (0, i, 0)),
            pl.BlockSpec((1, BM, D), lambda i: (1, i, 0)),
            pl.BlockSpec((2, D, D), lambda i: (0, 0, 0)),
        ],
        out_specs=pl.BlockSpec((BM, D), lambda i: (i, 0)),
        out_shape=jax.ShapeDtypeStruct((N_NODES, D), jnp.float32),
    )(g3, g3, w_stack)


def kernel(nodes, neigh_idx, features, detaching_weight):
    w_top = detaching_weight[:D]
    w_bot = detaching_weight[D:] * (1.0 / FILTER)
    w_stack = jnp.stack([w_top, w_bot])

    nodes32 = nodes.astype(jnp.int32)
    idx11 = jnp.concatenate(
        [nodes32[:, None], neigh_idx.astype(jnp.int32)], axis=1
    )
    pad_n = B_PAD - N_NODES
    pad_idx = (jnp.arange(pad_n * IDX_PER_ROW, dtype=jnp.int32) * 283 % N_NODES)
    idx_pad = jnp.concatenate(
        [idx11, pad_idx.reshape(pad_n, IDX_PER_ROW)], axis=0
    )
    idx_t = jnp.swapaxes(
        idx_pad.reshape(NW * GROUPS_PER_W, GROUP, IDX_PER_ROW), 1, 2
    )
    idx_hbm = idx_t.reshape(-1)

    gathered = _sc_gather(features, idx_hbm)
    return _fused_matmul(gathered, w_stack)


# speedup vs baseline: 1.9422x; 1.0131x over previous
import jax
import jax.numpy as jnp
from jax import lax
from jax.experimental import pallas as pl
from jax.experimental.pallas import tpu as pltpu
from jax.experimental.pallas import tpu_sc as plsc

N_NODES = 50000
D = 128
FILTER = 10

NW = 32
GROUP = 112
IDX_PER_ROW = 11
B_PAD = 50176
HALF = B_PAD // 2
ROWS_PER_W = HALF // NW
GROUPS_PER_W = ROWS_PER_W // GROUP
IDX_PER_GROUP = GROUP * IDX_PER_ROW

BM = 6272


def _make_sc_body(half):
    idx_base0 = half * HALF * IDX_PER_ROW

    def _sc_body(feat_hbm, idx_hbm, out_hbm, idx_v, self_v, acc_v, sem):
        wid = lax.axis_index("s") * 2 + lax.axis_index("c")
        zeros16 = jnp.zeros((16,), jnp.float32)

        def group_body(g, carry):
            idx_base = idx_base0 + (wid * GROUPS_PER_W + g) * IDX_PER_GROUP
            pltpu.sync_copy(idx_hbm.at[pl.ds(idx_base, IDX_PER_GROUP)], idx_v)

            def zero_body(r, c):
                for v in range(D // 16):
                    acc_v[r, pl.ds(v * 16, 16)] = zeros16
                return c

            lax.fori_loop(0, GROUP, zero_body, 0)

            copies = [
                pltpu.async_copy(
                    feat_hbm.at[idx_v.at[pl.ds(0, GROUP)]], self_v, sem
                )
            ] + [
                pltpu.async_copy(
                    feat_hbm.at[idx_v.at[pl.ds(j * GROUP, GROUP)]],
                    acc_v,
                    sem,
                    add=True,
                )
                for j in range(1, IDX_PER_ROW)
            ]
            for c in copies:
                c.wait()

            row0 = wid * ROWS_PER_W + g * GROUP
            pltpu.sync_copy(self_v, out_hbm.at[pl.ds(row0, GROUP)])
            pltpu.sync_copy(acc_v, out_hbm.at[pl.ds(HALF + row0, GROUP)])
            return carry

        lax.fori_loop(0, GROUPS_PER_W, group_body, 0)

    return _sc_body


def _sc_gather(features, idx_hbm, half):
    mesh = plsc.VectorSubcoreMesh(core_axis_name="c", subcore_axis_name="s")
    return pl.kernel(
        _make_sc_body(half),
        out_type=jax.ShapeDtypeStruct((2 * HALF, D), jnp.float32),
        mesh=mesh,
        scratch_types=[
            pltpu.VMEM((IDX_PER_GROUP,), jnp.int32),
            pltpu.VMEM((GROUP, D), jnp.float32),
            pltpu.VMEM((GROUP, D), jnp.float32),
            pltpu.SemaphoreType.DMA,
        ],
    )(features, idx_hbm)


def _matmul_body(s_ref, n_ref, w_ref, o_ref):
    ps = lax.dot_general(
        s_ref[0], w_ref[0],
        dimension_numbers=(((1,), (0,)), ((), ())),
        preferred_element_type=jnp.float32,
    )
    pn = lax.dot_general(
        n_ref[0], w_ref[1],
        dimension_numbers=(((1,), (0,)), ((), ())),
        preferred_element_type=jnp.float32,
    )
    o_ref[...] = jnp.maximum(ps + pn, 0.0)


def _matmul_body2(s_ref, n_ref, w_ref, prev_ref, o_ref):
    del prev_ref
    _matmul_body(s_ref, n_ref, w_ref, o_ref)


def _fused_matmul_h0(gathered, w_stack):
    nblk = HALF // BM
    g3 = gathered.reshape(2, HALF, D)
    return pl.pallas_call(
        _matmul_body,
        grid=(nblk,),
        in_specs=[
            pl.BlockSpec((1, BM, D), lambda i: (0, i, 0)),
            pl.BlockSpec((1, BM, D), lambda i: (1, i, 0)),
            pl.BlockSpec((2, D, D), lambda i: (0, 0, 0)),
        ],
        out_specs=pl.BlockSpec((BM, D), lambda i: (i, 0)),
        out_shape=jax.ShapeDtypeStruct((N_NODES, D), jnp.float32),
    )(g3, g3, w_stack)


def _fused_matmul_h1(gathered, w_stack, prev):
    nblk = HALF // BM
    base = HALF // BM
    g3 = gathered.reshape(2, HALF, D)
    return pl.pallas_call(
        _matmul_body2,
        grid=(nblk,),
        in_specs=[
            pl.BlockSpec((1, BM, D), lambda i: (0, i, 0)),
            pl.BlockSpec((1, BM, D), lambda i: (1, i, 0)),
            pl.BlockSpec((2, D, D), lambda i: (0, 0, 0)),
            pl.BlockSpec(memory_space=pl.ANY),
        ],
        out_specs=pl.BlockSpec((BM, D), lambda i: (i + base, 0)),
        out_shape=jax.ShapeDtypeStruct((N_NODES, D), jnp.float32),
        input_output_aliases={3: 0},
    )(g3, g3, w_stack, prev)


def kernel(nodes, neigh_idx, features, detaching_weight):
    w_top = detaching_weight[:D]
    w_bot = detaching_weight[D:] * (1.0 / FILTER)
    w_stack = jnp.stack([w_top, w_bot])

    nodes32 = nodes.astype(jnp.int32)
    idx11 = jnp.concatenate(
        [nodes32[:, None], neigh_idx.astype(jnp.int32)], axis=1
    )
    pad_n = B_PAD - N_NODES
    pad_idx = (jnp.arange(pad_n * IDX_PER_ROW, dtype=jnp.int32) * 283 % N_NODES)
    idx_pad = jnp.concatenate(
        [idx11, pad_idx.reshape(pad_n, IDX_PER_ROW)], axis=0
    )
    idx_t = jnp.swapaxes(
        idx_pad.reshape(2 * NW * GROUPS_PER_W, GROUP, IDX_PER_ROW), 1, 2
    )
    idx_hbm = idx_t.reshape(-1)

    g0 = _sc_gather(features, idx_hbm, 0)
    g1 = _sc_gather(features, idx_hbm, 1)
    o0 = _fused_matmul_h0(g0, w_stack)
    return _fused_matmul_h1(g1, w_stack, o0)


# speedup vs baseline: 1.9936x; 1.0265x over previous
import jax
import jax.numpy as jnp
from jax import lax
from jax.experimental import pallas as pl
from jax.experimental.pallas import tpu as pltpu
from jax.experimental.pallas import tpu_sc as plsc

N_NODES = 50000
D = 128
FILTER = 10

NW = 32
GROUP = 112
IDX_PER_ROW = 11
B_PAD = 50176
HALF = B_PAD // 2
ROWS_PER_W = HALF // NW
GROUPS_PER_W = ROWS_PER_W // GROUP
IDX_PER_GROUP = GROUP * IDX_PER_ROW

BM = 6272


def _sc_body(feat_hbm, idx_hbm, out_hbm, idx_v, self_v, acc_v, sem):
    wid = lax.axis_index("s") * 2 + lax.axis_index("c")
    zeros16 = jnp.zeros((16,), jnp.float32)

    def group_body(g, carry):
        idx_base = (wid * GROUPS_PER_W + g) * IDX_PER_GROUP
        pltpu.sync_copy(idx_hbm.at[pl.ds(idx_base, IDX_PER_GROUP)], idx_v)

        def zero_body(r, c):
            for v in range(D // 16):
                acc_v[r, pl.ds(v * 16, 16)] = zeros16
            return c

        lax.fori_loop(0, GROUP, zero_body, 0)

        copies = [
            pltpu.async_copy(
                feat_hbm.at[idx_v.at[pl.ds(0, GROUP)]], self_v, sem
            )
        ] + [
            pltpu.async_copy(
                feat_hbm.at[idx_v.at[pl.ds(j * GROUP, GROUP)]],
                acc_v,
                sem,
                add=True,
            )
            for j in range(1, IDX_PER_ROW)
        ]
        for c in copies:
            c.wait()

        row0 = wid * ROWS_PER_W + g * GROUP
        pltpu.sync_copy(self_v, out_hbm.at[pl.ds(row0, GROUP)])
        pltpu.sync_copy(acc_v, out_hbm.at[pl.ds(HALF + row0, GROUP)])
        return carry

    lax.fori_loop(0, GROUPS_PER_W, group_body, 0)


def _sc_gather(features, idx_hbm):
    mesh = plsc.VectorSubcoreMesh(core_axis_name="c", subcore_axis_name="s")
    return pl.kernel(
        _sc_body,
        out_type=jax.ShapeDtypeStruct((2 * HALF, D), jnp.float32),
        mesh=mesh,
        scratch_types=[
            pltpu.VMEM((IDX_PER_GROUP,), jnp.int32),
            pltpu.VMEM((GROUP, D), jnp.float32),
            pltpu.VMEM((GROUP, D), jnp.float32),
            pltpu.SemaphoreType.DMA,
        ],
    )(features, idx_hbm)


def _matmul_body(s_ref, n_ref, w_ref, o_ref):
    ps = lax.dot_general(
        s_ref[0], w_ref[0],
        dimension_numbers=(((1,), (0,)), ((), ())),
        preferred_element_type=jnp.float32,
    )
    pn = lax.dot_general(
        n_ref[0], w_ref[1],
        dimension_numbers=(((1,), (0,)), ((), ())),
        preferred_element_type=jnp.float32,
    )
    o_ref[...] = jnp.maximum(ps + pn, 0.0)


def _matmul_body2(s_ref, n_ref, w_ref, prev_ref, o_ref):
    del prev_ref
    _matmul_body(s_ref, n_ref, w_ref, o_ref)


def _fused_matmul_h0(gathered, w_stack):
    nblk = HALF // BM
    g3 = gathered.reshape(2, HALF, D)
    return pl.pallas_call(
        _matmul_body,
        grid=(nblk,),
        in_specs=[
            pl.BlockSpec((1, BM, D), lambda i: (0, i, 0)),
            pl.BlockSpec((1, BM, D), lambda i: (1, i, 0)),
            pl.BlockSpec((2, D, D), lambda i: (0, 0, 0)),
        ],
        out_specs=pl.BlockSpec((BM, D), lambda i: (i, 0)),
        out_shape=jax.ShapeDtypeStruct((N_NODES, D), jnp.float32),
    )(g3, g3, w_stack)


def _fused_matmul_h1(gathered, w_stack, prev):
    nblk = HALF // BM
    base = HALF // BM
    g3 = gathered.reshape(2, HALF, D)
    return pl.pallas_call(
        _matmul_body2,
        grid=(nblk,),
        in_specs=[
            pl.BlockSpec((1, BM, D), lambda i: (0, i, 0)),
            pl.BlockSpec((1, BM, D), lambda i: (1, i, 0)),
            pl.BlockSpec((2, D, D), lambda i: (0, 0, 0)),
            pl.BlockSpec(memory_space=pl.ANY),
        ],
        out_specs=pl.BlockSpec((BM, D), lambda i: (i + base, 0)),
        out_shape=jax.ShapeDtypeStruct((N_NODES, D), jnp.float32),
        input_output_aliases={3: 0},
    )(g3, g3, w_stack, prev)


def kernel(nodes, neigh_idx, features, detaching_weight):
    w_top = detaching_weight[:D]
    w_bot = detaching_weight[D:] * (1.0 / FILTER)
    w_stack = jnp.stack([w_top, w_bot])

    nodes32 = nodes.astype(jnp.int32)
    idx11 = jnp.concatenate(
        [nodes32[:, None], neigh_idx.astype(jnp.int32)], axis=1
    )
    pad_n = B_PAD - N_NODES
    pad_idx = (jnp.arange(pad_n * IDX_PER_ROW, dtype=jnp.int32) * 283 % N_NODES)
    idx_pad = jnp.concatenate(
        [idx11, pad_idx.reshape(pad_n, IDX_PER_ROW)], axis=0
    )

    def half_idx(h):
        rows = lax.slice_in_dim(idx_pad, h * HALF, (h + 1) * HALF, axis=0)
        return jnp.swapaxes(
            rows.reshape(NW * GROUPS_PER_W, GROUP, IDX_PER_ROW), 1, 2
        ).reshape(-1)

    idx0 = half_idx(0)
    g0 = _sc_gather(features, idx0)
    idx1 = half_idx(1)
    g1 = _sc_gather(features, idx1)
    o0 = _fused_matmul_h0(g0, w_stack)
    return _fused_matmul_h1(g1, w_stack, o0)


# speedup vs baseline: 2.0541x; 1.0303x over previous
import jax
import jax.numpy as jnp
from jax import lax
from jax.experimental import pallas as pl
from jax.experimental.pallas import tpu as pltpu
from jax.experimental.pallas import tpu_sc as plsc

N_NODES = 50000
D = 128
FILTER = 10

NW = 32
GROUP = 112
IDX_PER_ROW = 11
B_PAD = 50176
HALF = B_PAD // 2
ROWS_PER_W = HALF // NW
GROUPS_PER_W = ROWS_PER_W // GROUP
IDX_PER_GROUP = GROUP * IDX_PER_ROW

BM = 6272


def _sc_body(feat_hbm, idx_hbm, out_hbm, idx_v, self_v, acc_v, sem):
    wid = lax.axis_index("s") * 2 + lax.axis_index("c")
    zeros16 = jnp.zeros((16,), jnp.float32)

    def group_body(g, carry):
        idx_base = (wid * GROUPS_PER_W + g) * IDX_PER_GROUP
        idx_cp = pltpu.async_copy(
            idx_hbm.at[pl.ds(idx_base, IDX_PER_GROUP)], idx_v, sem
        )

        def zero_body(r, c):
            for v in range(D // 16):
                acc_v[r, pl.ds(v * 16, 16)] = zeros16
            return c

        lax.fori_loop(0, GROUP, zero_body, 0)
        idx_cp.wait()

        copies = [
            pltpu.async_copy(
                feat_hbm.at[idx_v.at[pl.ds(0, GROUP)]], self_v, sem
            )
        ] + [
            pltpu.async_copy(
                feat_hbm.at[idx_v.at[pl.ds(j * GROUP, GROUP)]],
                acc_v,
                sem,
                add=True,
            )
            for j in range(1, IDX_PER_ROW)
        ]
        for c in copies:
            c.wait()

        row0 = wid * ROWS_PER_W + g * GROUP
        pltpu.sync_copy(self_v, out_hbm.at[pl.ds(row0, GROUP)])
        pltpu.sync_copy(acc_v, out_hbm.at[pl.ds(HALF + row0, GROUP)])
        return carry

    lax.fori_loop(0, GROUPS_PER_W, group_body, 0)


def _sc_gather(features, idx_hbm):
    mesh = plsc.VectorSubcoreMesh(core_axis_name="c", subcore_axis_name="s")
    return pl.kernel(
        _sc_body,
        out_type=jax.ShapeDtypeStruct((2 * HALF, D), jnp.float32),
        mesh=mesh,
        scratch_types=[
            pltpu.VMEM((IDX_PER_GROUP,), jnp.int32),
            pltpu.VMEM((GROUP, D), jnp.float32),
            pltpu.VMEM((GROUP, D), jnp.float32),
            pltpu.SemaphoreType.DMA,
        ],
    )(features, idx_hbm)


def _matmul_body(s_ref, n_ref, w_ref, o_ref):
    ps = lax.dot_general(
        s_ref[0], w_ref[0],
        dimension_numbers=(((1,), (0,)), ((), ())),
        preferred_element_type=jnp.float32,
    )
    pn = lax.dot_general(
        n_ref[0], w_ref[1],
        dimension_numbers=(((1,), (0,)), ((), ())),
        preferred_element_type=jnp.float32,
    )
    o_ref[...] = jnp.maximum(ps + pn, 0.0)


def _matmul_body2(s_ref, n_ref, w_ref, prev_ref, o_ref):
    del prev_ref
    _matmul_body(s_ref, n_ref, w_ref, o_ref)


def _fused_matmul_h0(gathered, w_stack):
    nblk = HALF // BM
    g3 = gathered.reshape(2, HALF, D)
    return pl.pallas_call(
        _matmul_body,
        grid=(nblk,),
        in_specs=[
            pl.BlockSpec((1, BM, D), lambda i: (0, i, 0)),
            pl.BlockSpec((1, BM, D), lambda i: (1, i, 0)),
            pl.BlockSpec((2, D, D), lambda i: (0, 0, 0)),
        ],
        out_specs=pl.BlockSpec((BM, D), lambda i: (i, 0)),
        out_shape=jax.ShapeDtypeStruct((N_NODES, D), jnp.float32),
    )(g3, g3, w_stack)


def _fused_matmul_h1(gathered, w_stack, prev):
    nblk = HALF // BM
    base = HALF // BM
    g3 = gathered.reshape(2, HALF, D)
    return pl.pallas_call(
        _matmul_body2,
        grid=(nblk,),
        in_specs=[
            pl.BlockSpec((1, BM, D), lambda i: (0, i, 0)),
            pl.BlockSpec((1, BM, D), lambda i: (1, i, 0)),
            pl.BlockSpec((2, D, D), lambda i: (0, 0, 0)),
            pl.BlockSpec(memory_space=pl.ANY),
        ],
        out_specs=pl.BlockSpec((BM, D), lambda i: (i + base, 0)),
        out_shape=jax.ShapeDtypeStruct((N_NODES, D), jnp.float32),
        input_output_aliases={3: 0},
    )(g3, g3, w_stack, prev)


def kernel(nodes, neigh_idx, features, detaching_weight):
    w_top = detaching_weight[:D]
    w_bot = detaching_weight[D:] * (1.0 / FILTER)
    w_stack = jnp.stack([w_top, w_bot])

    nodes32 = nodes.astype(jnp.int32)
    idx11 = jnp.concatenate(
        [nodes32[:, None], neigh_idx.astype(jnp.int32)], axis=1
    )
    pad_n = B_PAD - N_NODES
    pad_idx = (jnp.arange(pad_n * IDX_PER_ROW, dtype=jnp.int32) * 283 % N_NODES)
    idx_pad = jnp.concatenate(
        [idx11, pad_idx.reshape(pad_n, IDX_PER_ROW)], axis=0
    )

    def half_idx(h):
        rows = lax.slice_in_dim(idx_pad, h * HALF, (h + 1) * HALF, axis=0)
        return jnp.swapaxes(
            rows.reshape(NW * GROUPS_PER_W, GROUP, IDX_PER_ROW), 1, 2
        ).reshape(-1)

    idx0 = half_idx(0)
    g0 = _sc_gather(features, idx0)
    idx1 = half_idx(1)
    g1 = _sc_gather(features, idx1)
    o0 = _fused_matmul_h0(g0, w_stack)
    return _fused_matmul_h1(g1, w_stack, o0)
